# baseline (speedup 1.0000x reference)
import functools

import jax
import jax.numpy as jnp
from jax import lax
from jax.experimental import pallas as pl
from jax.experimental.pallas import tpu as pltpu
from jax.experimental.pallas import tpu_sc as plsc

N = 10000
NP = 10240
E = 320000
D_IN = 128
D_H = 128
D_OUT = 13
D_PAD = 16

NC = 2
NS = 16
NT = NC * NS
EPT = E // NT
K = 128
EPTP = 10240
CH = EPTP // K
CHH = CH // 2
RPT = NP // NS
GARB = NP - 1

BR = 1024

_mesh = plsc.VectorSubcoreMesh(core_axis_name="c", subcore_axis_name="s")


KE = 80
NCK = EPT // KE


@functools.partial(
    pl.kernel,
    out_type=jax.ShapeDtypeStruct((NC, NP, D_H), jnp.float32),
    mesh=_mesh,
    scratch_types=[
        pltpu.VMEM_SHARED((NP, D_H), jnp.float32),
        pltpu.VMEM((KE,), jnp.int32),
        pltpu.VMEM((KE,), jnp.int32),
        pltpu.VMEM((KE,), jnp.int32),
        pltpu.VMEM((KE,), jnp.int32),
        pltpu.VMEM((KE,), jnp.int32),
        pltpu.VMEM((KE,), jnp.int32),
        pltpu.VMEM((KE, D_H), jnp.float32),
        pltpu.VMEM((KE, D_H), jnp.float32),
        pltpu.VMEM((KE, D_H), jnp.float32),
        pltpu.SemaphoreType.DMA,
        pltpu.SemaphoreType.DMA,
        pltpu.SemaphoreType.DMA,
        pltpu.SemaphoreType.DMA,
        pltpu.SemaphoreType.DMA,
        pltpu.SemaphoreType.DMA,
        pltpu.SemaphoreType.DMA,
        pltpu.SemaphoreType.DMA,
        pltpu.SemaphoreType.DMA,
        pltpu.SemaphoreType.DMA,
        pltpu.SemaphoreType.DMA,
        pltpu.SemaphoreType.DMA,
    ],
)
def _scatter_pass(y_hbm, src_hbm, dst_hbm, out_hbm, z_sh,
                  sb0, db0, sb1, db1, sb2, db2, rows0, rows1, rows2,
                  sis0, sid0, sis1, sid1, sis2, sid2, g0, g1, g2,
                  s0, s1, s2):
    cid = lax.axis_index("c")
    sid = lax.axis_index("s")
    ebase = (cid * NS + sid) * EPT

    zero16 = jnp.zeros((16,), jnp.float32)

    @pl.loop(0, KE)
    def _zrow(r):
        @pl.loop(0, D_H // 16)
        def _zcol(j):
            rows0[r, pl.ds(j * 16, 16)] = zero16

    @pl.loop(0, RPT // KE)
    def _zinit(k):
        pltpu.sync_copy(rows0, z_sh.at[pl.ds(sid * RPT + k * KE, KE)])

    def _idxload(c, sb, db, ss, sd):
        pltpu.async_copy(src_hbm.at[pl.ds(ebase + c * KE, KE)], sb, ss)
        pltpu.async_copy(dst_hbm.at[pl.ds(ebase + c * KE, KE)], db, sd)

    def _wait_idx(sb, db, ss, sd):
        pltpu.make_async_copy(src_hbm.at[pl.ds(0, KE)], sb, ss).wait()
        pltpu.make_async_copy(src_hbm.at[pl.ds(0, KE)], db, sd).wait()

    def _gather(sb, rows, sem):
        pltpu.async_copy(y_hbm.at[sb], rows, sem)

    def _drain(rows, sem):
        pltpu.make_async_copy(y_hbm.at[pl.ds(0, KE)], rows, sem).wait()

    def _scatter(db, rows, sem):
        pltpu.async_copy(rows, z_sh.at[db], sem, add=True)

    def _sdrain(rows, sem):
        pltpu.make_async_copy(rows, z_sh.at[pl.ds(0, KE)], sem).wait()

    _idxload(0, sb0, db0, sis0, sid0)
    _idxload(1, sb1, db1, sis1, sid1)
    _idxload(2, sb2, db2, sis2, sid2)
    _wait_idx(sb0, db0, sis0, sid0)
    _gather(sb0, rows0, g0)
    _wait_idx(sb1, db1, sis1, sid1)
    _gather(sb1, rows1, g1)
    _wait_idx(sb2, db2, sis2, sid2)
    _gather(sb2, rows2, g2)

    plsc.subcore_barrier()

    @pl.loop(0, NCK - 3, step=3)
    def _chunk(c):
        _drain(rows0, g0)
        _scatter(db0, rows0, s0)
        _drain(rows1, g1)
        _scatter(db1, rows1, s1)
        _sdrain(rows0, s0)
        _idxload(jnp.minimum(c + 3, NCK - 1), sb0, db0, sis0, sid0)
        _wait_idx(sb0, db0, sis0, sid0)
        _gather(sb0, rows0, g0)
        _drain(rows2, g2)
        _scatter(db2, rows2, s2)
        _sdrain(rows1, s1)
        _idxload(jnp.minimum(c + 4, NCK - 1), sb1, db1, sis1, sid1)
        _wait_idx(sb1, db1, sis1, sid1)
        _gather(sb1, rows1, g1)
        _sdrain(rows2, s2)
        _idxload(jnp.minimum(c + 5, NCK - 1), sb2, db2, sis2, sid2)
        _wait_idx(sb2, db2, sis2, sid2)
        _gather(sb2, rows2, g2)

    _drain(rows0, g0)
    _scatter(db0, rows0, s0)
    _drain(rows1, g1)
    _scatter(db1, rows1, s1)
    _drain(rows2, g2)
    _sdrain(rows0, s0)
    _sdrain(rows1, s1)

    plsc.subcore_barrier()
    pltpu.sync_copy(z_sh.at[pl.ds(sid * RPT, RPT)],
                    out_hbm.at[cid, pl.ds(sid * RPT, RPT)])


@functools.partial(
    pl.kernel,
    out_type=jax.ShapeDtypeStruct((NC, NP, D_H), jnp.float32),
    mesh=_mesh,
    scratch_types=[
        pltpu.VMEM_SHARED((NP, D_H), jnp.float32),
        pltpu.VMEM((CHH, K), jnp.int32),
        pltpu.VMEM((K, D_H), jnp.float32),
    ],
)
def _deg_pass(src_hbm, out_hbm, z_sh, src_t, ones_v):
    cid = lax.axis_index("c")
    sid = lax.axis_index("s")
    tid = cid * NS + sid

    zero16 = jnp.zeros((16,), jnp.float32)
    one16 = jnp.ones((16,), jnp.float32)

    @pl.loop(0, K)
    def _zrow(r):
        @pl.loop(0, D_H // 16)
        def _zcol(j):
            ones_v[r, pl.ds(j * 16, 16)] = zero16

    @pl.loop(0, RPT // K)
    def _zinit(k):
        pltpu.sync_copy(ones_v, z_sh.at[pl.ds(sid * RPT + k * K, K)])

    @pl.loop(0, K)
    def _orow(r):
        @pl.loop(0, D_H // 16)
        def _ocol(j):
            ones_v[r, pl.ds(j * 16, 16)] = one16

    plsc.subcore_barrier()

    for h in range(CH // CHH):
        pltpu.sync_copy(src_hbm.at[tid, pl.ds(h * CHH, CHH)], src_t)

        @pl.loop(0, CHH)
        def _chunk(c):
            pltpu.sync_copy(ones_v, z_sh.at[src_t.at[c]], add=True)

    plsc.subcore_barrier()
    pltpu.sync_copy(z_sh.at[pl.ds(sid * RPT, RPT)],
                    out_hbm.at[cid, pl.ds(sid * RPT, RPT)])


def _dot_t(a, w):
    return lax.dot_general(a, w, (((1,), (1,)), ((), ())),
                           preferred_element_type=jnp.float32)


def _tc_first(d0, d1, x, w, b):
    def body(d0_ref, d1_ref, x_ref, w_ref, b_ref, dinv_ref, y_ref):
        deg = 1.0 + d0_ref[:, 0:1] + d1_ref[:, 0:1]
        dinv = lax.rsqrt(deg)
        dinv_ref[...] = dinv
        y_ref[...] = dinv * (_dot_t(x_ref[...], w_ref[...]) + b_ref[...])

    return pl.pallas_call(
        body,
        grid=(NP // BR,),
        in_specs=[
            pl.BlockSpec((BR, D_H), lambda i: (i, 0)),
            pl.BlockSpec((BR, D_H), lambda i: (i, 0)),
            pl.BlockSpec((BR, D_IN), lambda i: (i, 0)),
            pl.BlockSpec((D_H, D_IN), lambda i: (0, 0)),
            pl.BlockSpec((1, D_H), lambda i: (0, 0)),
        ],
        out_specs=[
            pl.BlockSpec((BR, 1), lambda i: (i, 0)),
            pl.BlockSpec((BR, D_H), lambda i: (i, 0)),
        ],
        out_shape=[
            jax.ShapeDtypeStruct((NP, 1), jnp.float32),
            jax.ShapeDtypeStruct((NP, D_H), jnp.float32),
        ],
    )(d0, d1, x, w, b)


def _tc_mid(z0, z1, y_prev, dinv, w, b):
    def body(z0_ref, z1_ref, yp_ref, dinv_ref, w_ref, b_ref, y_ref):
        dinv = dinv_ref[...]
        h = jnp.maximum(dinv * (z0_ref[...] + z1_ref[...] + yp_ref[...]), 0.0)
        y_ref[...] = dinv * (_dot_t(h, w_ref[...]) + b_ref[...])

    return pl.pallas_call(
        body,
        grid=(NP // BR,),
        in_specs=[
            pl.BlockSpec((BR, D_H), lambda i: (i, 0)),
            pl.BlockSpec((BR, D_H), lambda i: (i, 0)),
            pl.BlockSpec((BR, D_H), lambda i: (i, 0)),
            pl.BlockSpec((BR, 1), lambda i: (i, 0)),
            pl.BlockSpec((D_H, D_H), lambda i: (0, 0)),
            pl.BlockSpec((1, D_H), lambda i: (0, 0)),
        ],
        out_specs=pl.BlockSpec((BR, D_H), lambda i: (i, 0)),
        out_shape=jax.ShapeDtypeStruct((NP, D_H), jnp.float32),
    )(z0, z1, y_prev, dinv, w, b)


def _tc_final(z0, z1, y_prev, dinv, r_pad):
    def body(z0_ref, z1_ref, yp_ref, dinv_ref, r_ref, out_ref):
        s = jax.nn.sigmoid(
            dinv_ref[...] * (z0_ref[...] + z1_ref[...] + yp_ref[...]))
        cols = []
        for i in range(D_PAD):
            ri = r_ref[i, :]
            cols.append(jnp.max(ri[None, :] * s, axis=1, keepdims=True))
        out_ref[...] = jnp.concatenate(cols, axis=1)

    return pl.pallas_call(
        body,
        grid=(NP // BR,),
        in_specs=[
            pl.BlockSpec((BR, D_H), lambda i: (i, 0)),
            pl.BlockSpec((BR, D_H), lambda i: (i, 0)),
            pl.BlockSpec((BR, D_H), lambda i: (i, 0)),
            pl.BlockSpec((BR, 1), lambda i: (i, 0)),
            pl.BlockSpec((D_PAD, D_H), lambda i: (0, 0)),
        ],
        out_specs=pl.BlockSpec((BR, D_PAD), lambda i: (i, 0)),
        out_shape=jax.ShapeDtypeStruct((NP, D_PAD), jnp.float32),
    )(z0, z1, y_prev, dinv, r_pad)


@jax.jit
def kernel(x, edge_index, W0, b0, W1, b1, W2, b2, R):
    xp = jnp.zeros((NP, D_IN), jnp.float32).at[:N].set(x)

    W2p = jnp.zeros((D_H, D_H), jnp.float32).at[:D_OUT].set(W2)
    b2p = jnp.zeros((D_H,), jnp.float32).at[:D_OUT].set(b2)
    Rp = jnp.zeros((D_PAD, D_H), jnp.float32).at[:D_OUT, :D_OUT].set(R)

    pad = ((0, 0), (0, EPTP - EPT))
    src_ids = jnp.pad(edge_index[0].reshape(NT, EPT), pad,
                      constant_values=GARB).reshape(NT, CH, K)
    dst_ids = jnp.pad(edge_index[1].reshape(NT, EPT), pad,
                      constant_values=GARB).reshape(NT, CH, K)
    src_flat = edge_index[0]
    dst_flat = edge_index[1]

    degp = _deg_pass(src_ids)
    dinv, y0 = _tc_first(degp[0], degp[1], xp, W0,
                         b0.reshape(1, D_H))
    z0 = _scatter_pass(y0, src_flat, dst_flat)
    y1 = _tc_mid(z0[0], z0[1], y0, dinv, W1, b1.reshape(1, D_H))
    z1 = _scatter_pass(y1, src_flat, dst_flat)
    y2 = _tc_mid(z1[0], z1[1], y1, dinv, W2p, b2p.reshape(1, D_H))
    z2 = _scatter_pass(y2, src_flat, dst_flat)
    out = _tc_final(z2[0], z2[1], y2, dinv, Rp)
    return out[:N, :D_OUT]


# speedup vs baseline: 1.0132x; 1.0132x over previous
import functools

import jax
import jax.numpy as jnp
from jax import lax
from jax.experimental import pallas as pl
from jax.experimental.pallas import tpu as pltpu
from jax.experimental.pallas import tpu_sc as plsc

N = 10000
NP = 10240
E = 320000
D_IN = 128
D_H = 128
D_OUT = 13
D_PAD = 16

NC = 2
NS = 16
NT = NC * NS
EPT = E // NT
K = 128
EPTP = 10240
CH = EPTP // K
CHH = CH // 2
RPT = NP // NS
GARB = NP - 1

BR = 1024

_mesh = plsc.VectorSubcoreMesh(core_axis_name="c", subcore_axis_name="s")


KE = 80
NCK = EPT // KE


@functools.partial(
    pl.kernel,
    out_type=jax.ShapeDtypeStruct((NC, NP, D_H), jnp.float32),
    mesh=_mesh,
    scratch_types=[
        pltpu.VMEM_SHARED((NP, D_H), jnp.float32),
        pltpu.VMEM((KE,), jnp.int32),
        pltpu.VMEM((KE,), jnp.int32),
        pltpu.VMEM((KE,), jnp.int32),
        pltpu.VMEM((KE,), jnp.int32),
        pltpu.VMEM((KE,), jnp.int32),
        pltpu.VMEM((KE,), jnp.int32),
        pltpu.VMEM((KE, D_H), jnp.float32),
        pltpu.VMEM((KE, D_H), jnp.float32),
        pltpu.VMEM((KE, D_H), jnp.float32),
        pltpu.SemaphoreType.DMA,
        pltpu.SemaphoreType.DMA,
        pltpu.SemaphoreType.DMA,
        pltpu.SemaphoreType.DMA,
        pltpu.SemaphoreType.DMA,
        pltpu.SemaphoreType.DMA,
        pltpu.SemaphoreType.DMA,
        pltpu.SemaphoreType.DMA,
        pltpu.SemaphoreType.DMA,
    ],
)
def _scatter_pass(y_hbm, src_hbm, dst_hbm, out_hbm, z_sh,
                  sb0, db0, sb1, db1, sb2, db2, rows0, rows1, rows2,
                  sis0, sid0, sis1, sid1, sis2, sid2, g0, g1, g2):
    cid = lax.axis_index("c")
    sid = lax.axis_index("s")
    ebase = (cid * NS + sid) * EPT

    zero16 = jnp.zeros((16,), jnp.float32)

    @pl.loop(0, KE)
    def _zrow(r):
        @pl.loop(0, D_H // 16)
        def _zcol(j):
            rows0[r, pl.ds(j * 16, 16)] = zero16

    @pl.loop(0, RPT // KE)
    def _zinit(k):
        pltpu.sync_copy(rows0, z_sh.at[pl.ds(sid * RPT + k * KE, KE)])

    def _idxload(c, sb, db, ss, sd):
        pltpu.async_copy(src_hbm.at[pl.ds(ebase + c * KE, KE)], sb, ss)
        pltpu.async_copy(dst_hbm.at[pl.ds(ebase + c * KE, KE)], db, sd)

    def _wait_idx(sb, db, ss, sd):
        pltpu.make_async_copy(src_hbm.at[pl.ds(0, KE)], sb, ss).wait()
        pltpu.make_async_copy(src_hbm.at[pl.ds(0, KE)], db, sd).wait()

    def _gather(sb, rows, sem):
        pltpu.async_copy(y_hbm.at[sb], rows, sem)

    def _drain(rows, sem):
        pltpu.make_async_copy(y_hbm.at[pl.ds(0, KE)], rows, sem).wait()

    def _scatter(db, rows):
        pltpu.sync_copy(rows, z_sh.at[db], add=True)

    _idxload(0, sb0, db0, sis0, sid0)
    _idxload(1, sb1, db1, sis1, sid1)
    _idxload(2, sb2, db2, sis2, sid2)
    _wait_idx(sb0, db0, sis0, sid0)
    _gather(sb0, rows0, g0)
    _wait_idx(sb1, db1, sis1, sid1)
    _gather(sb1, rows1, g1)

    plsc.subcore_barrier()

    @pl.loop(0, NCK - 4, step=3)
    def _chunk(c):
        _wait_idx(sb2, db2, sis2, sid2)
        _gather(sb2, rows2, g2)
        _drain(rows0, g0)
        _scatter(db0, rows0)
        _idxload(c + 3, sb0, db0, sis0, sid0)
        _drain(rows1, g1)
        _scatter(db1, rows1)
        _idxload(c + 4, sb1, db1, sis1, sid1)
        _wait_idx(sb0, db0, sis0, sid0)
        _gather(sb0, rows0, g0)
        _drain(rows2, g2)
        _scatter(db2, rows2)
        _idxload(jnp.minimum(c + 5, NCK - 1), sb2, db2, sis2, sid2)
        _wait_idx(sb1, db1, sis1, sid1)
        _gather(sb1, rows1, g1)

    _drain(rows0, g0)
    _scatter(db0, rows0)
    _drain(rows1, g1)
    _scatter(db1, rows1)
    _wait_idx(sb2, db2, sis2, sid2)

    plsc.subcore_barrier()
    pltpu.sync_copy(z_sh.at[pl.ds(sid * RPT, RPT)],
                    out_hbm.at[cid, pl.ds(sid * RPT, RPT)])


@functools.partial(
    pl.kernel,
    out_type=jax.ShapeDtypeStruct((NC, NP, D_H), jnp.float32),
    mesh=_mesh,
    scratch_types=[
        pltpu.VMEM_SHARED((NP, D_H), jnp.float32),
        pltpu.VMEM((CHH, K), jnp.int32),
        pltpu.VMEM((K, D_H), jnp.float32),
    ],
)
def _deg_pass(src_hbm, out_hbm, z_sh, src_t, ones_v):
    cid = lax.axis_index("c")
    sid = lax.axis_index("s")
    tid = cid * NS + sid

    zero16 = jnp.zeros((16,), jnp.float32)
    one16 = jnp.ones((16,), jnp.float32)

    @pl.loop(0, K)
    def _zrow(r):
        @pl.loop(0, D_H // 16)
        def _zcol(j):
            ones_v[r, pl.ds(j * 16, 16)] = zero16

    @pl.loop(0, RPT // K)
    def _zinit(k):
        pltpu.sync_copy(ones_v, z_sh.at[pl.ds(sid * RPT + k * K, K)])

    @pl.loop(0, K)
    def _orow(r):
        @pl.loop(0, D_H // 16)
        def _ocol(j):
            ones_v[r, pl.ds(j * 16, 16)] = one16

    plsc.subcore_barrier()

    for h in range(CH // CHH):
        pltpu.sync_copy(src_hbm.at[tid, pl.ds(h * CHH, CHH)], src_t)

        @pl.loop(0, CHH)
        def _chunk(c):
            pltpu.sync_copy(ones_v, z_sh.at[src_t.at[c]], add=True)

    plsc.subcore_barrier()
    pltpu.sync_copy(z_sh.at[pl.ds(sid * RPT, RPT)],
                    out_hbm.at[cid, pl.ds(sid * RPT, RPT)])


def _dot_t(a, w):
    return lax.dot_general(a, w, (((1,), (1,)), ((), ())),
                           preferred_element_type=jnp.float32)


def _tc_first(d0, d1, x, w, b):
    def body(d0_ref, d1_ref, x_ref, w_ref, b_ref, dinv_ref, y_ref):
        deg = 1.0 + d0_ref[:, 0:1] + d1_ref[:, 0:1]
        dinv = lax.rsqrt(deg)
        dinv_ref[...] = dinv
        y_ref[...] = dinv * (_dot_t(x_ref[...], w_ref[...]) + b_ref[...])

    return pl.pallas_call(
        body,
        grid=(NP // BR,),
        in_specs=[
            pl.BlockSpec((BR, D_H), lambda i: (i, 0)),
            pl.BlockSpec((BR, D_H), lambda i: (i, 0)),
            pl.BlockSpec((BR, D_IN), lambda i: (i, 0)),
            pl.BlockSpec((D_H, D_IN), lambda i: (0, 0)),
            pl.BlockSpec((1, D_H), lambda i: (0, 0)),
        ],
        out_specs=[
            pl.BlockSpec((BR, 1), lambda i: (i, 0)),
            pl.BlockSpec((BR, D_H), lambda i: (i, 0)),
        ],
        out_shape=[
            jax.ShapeDtypeStruct((NP, 1), jnp.float32),
            jax.ShapeDtypeStruct((NP, D_H), jnp.float32),
        ],
    )(d0, d1, x, w, b)


def _tc_mid(z0, z1, y_prev, dinv, w, b):
    def body(z0_ref, z1_ref, yp_ref, dinv_ref, w_ref, b_ref, y_ref):
        dinv = dinv_ref[...]
        h = jnp.maximum(dinv * (z0_ref[...] + z1_ref[...] + yp_ref[...]), 0.0)
        y_ref[...] = dinv * (_dot_t(h, w_ref[...]) + b_ref[...])

    return pl.pallas_call(
        body,
        grid=(NP // BR,),
        in_specs=[
            pl.BlockSpec((BR, D_H), lambda i: (i, 0)),
            pl.BlockSpec((BR, D_H), lambda i: (i, 0)),
            pl.BlockSpec((BR, D_H), lambda i: (i, 0)),
            pl.BlockSpec((BR, 1), lambda i: (i, 0)),
            pl.BlockSpec((D_H, D_H), lambda i: (0, 0)),
            pl.BlockSpec((1, D_H), lambda i: (0, 0)),
        ],
        out_specs=pl.BlockSpec((BR, D_H), lambda i: (i, 0)),
        out_shape=jax.ShapeDtypeStruct((NP, D_H), jnp.float32),
    )(z0, z1, y_prev, dinv, w, b)


def _tc_final(z0, z1, y_prev, dinv, r_pad):
    def body(z0_ref, z1_ref, yp_ref, dinv_ref, r_ref, out_ref):
        s = jax.nn.sigmoid(
            dinv_ref[...] * (z0_ref[...] + z1_ref[...] + yp_ref[...]))
        cols = []
        for i in range(D_PAD):
            ri = r_ref[i, :]
            cols.append(jnp.max(ri[None, :] * s, axis=1, keepdims=True))
        out_ref[...] = jnp.concatenate(cols, axis=1)

    return pl.pallas_call(
        body,
        grid=(NP // BR,),
        in_specs=[
            pl.BlockSpec((BR, D_H), lambda i: (i, 0)),
            pl.BlockSpec((BR, D_H), lambda i: (i, 0)),
            pl.BlockSpec((BR, D_H), lambda i: (i, 0)),
            pl.BlockSpec((BR, 1), lambda i: (i, 0)),
            pl.BlockSpec((D_PAD, D_H), lambda i: (0, 0)),
        ],
        out_specs=pl.BlockSpec((BR, D_PAD), lambda i: (i, 0)),
        out_shape=jax.ShapeDtypeStruct((NP, D_PAD), jnp.float32),
    )(z0, z1, y_prev, dinv, r_pad)


@jax.jit
def kernel(x, edge_index, W0, b0, W1, b1, W2, b2, R):
    xp = jnp.zeros((NP, D_IN), jnp.float32).at[:N].set(x)

    W2p = jnp.zeros((D_H, D_H), jnp.float32).at[:D_OUT].set(W2)
    b2p = jnp.zeros((D_H,), jnp.float32).at[:D_OUT].set(b2)
    Rp = jnp.zeros((D_PAD, D_H), jnp.float32).at[:D_OUT, :D_OUT].set(R)

    pad = ((0, 0), (0, EPTP - EPT))
    src_ids = jnp.pad(edge_index[0].reshape(NT, EPT), pad,
                      constant_values=GARB).reshape(NT, CH, K)
    dst_ids = jnp.pad(edge_index[1].reshape(NT, EPT), pad,
                      constant_values=GARB).reshape(NT, CH, K)
    src_flat = edge_index[0]
    dst_flat = edge_index[1]

    degp = _deg_pass(src_ids)
    dinv, y0 = _tc_first(degp[0], degp[1], xp, W0,
                         b0.reshape(1, D_H))
    z0 = _scatter_pass(y0, src_flat, dst_flat)
    y1 = _tc_mid(z0[0], z0[1], y0, dinv, W1, b1.reshape(1, D_H))
    z1 = _scatter_pass(y1, src_flat, dst_flat)
    y2 = _tc_mid(z1[0], z1[1], y1, dinv, W2p, b2p.reshape(1, D_H))
    z2 = _scatter_pass(y2, src_flat, dst_flat)
    out = _tc_final(z2[0], z2[1], y2, dinv, Rp)
    return out[:N, :D_OUT]
